# baseline (speedup 1.0000x reference)
import functools

import jax
import jax.numpy as jnp
from jax import lax
from jax.experimental import pallas as pl
from jax.experimental.pallas import tpu as pltpu
from jax.experimental.pallas import tpu_sc as plsc

Q, K, D, DT = 1024, 100000, 1024, 300
GROUPS, MID, OUT = 64, 4096, 1024
KBLK = 2000
NBLK = K // KBLK
NSTREAMS = 2
NSTEPS = NBLK // NSTREAMS


def _argmax_body(p_ref, *refs):
    g_refs = refs[:NSTREAMS]
    idx_ref = refs[NSTREAMS]
    maxv, argv = refs[NSTREAMS + 1:]
    i = pl.program_id(0)

    @pl.when(i == 0)
    def _():
        maxv[...] = jnp.full_like(maxv[...], -jnp.inf)
        argv[...] = jnp.zeros_like(argv[...])

    p = p_ref[...]
    best_m = None
    best_a = None
    for j, g_ref in enumerate(g_refs):
        g = g_ref[...]
        ss = jnp.sum(g * g, axis=1, keepdims=True)
        gn = g * (1.0 / (jnp.sqrt(ss) + 1e-8))
        s = lax.dot_general(p, gn, (((1,), (1,)), ((), ())),
                            preferred_element_type=jnp.float32)
        col = lax.broadcasted_iota(jnp.int32, s.shape, 1)
        m = jnp.max(s, axis=1, keepdims=True)
        cand = jnp.where(s == m, col, K)
        a = (jnp.min(cand, axis=1, keepdims=True)
             + (i * NSTREAMS + j) * KBLK)
        if best_m is None:
            best_m, best_a = m, a
        else:
            take = m > best_m
            best_m = jnp.where(take, m, best_m)
            best_a = jnp.where(take, a, best_a)
    upd = best_m > maxv[...]
    maxv[...] = jnp.where(upd, best_m, maxv[...])
    argv[...] = jnp.where(upd, best_a, argv[...])
    idx_ref[...] = argv[...]


def _top1_index(p_head, g_head):
    g_specs = [
        pl.BlockSpec((KBLK, D), lambda i, j=j: (NSTREAMS * i + j, 0))
        for j in range(NSTREAMS)
    ]
    return pl.pallas_call(
        _argmax_body,
        grid=(NSTEPS,),
        in_specs=[pl.BlockSpec((Q, D), lambda i: (0, 0))] + g_specs,
        out_specs=pl.BlockSpec((Q, 1), lambda i: (0, 0)),
        out_shape=jax.ShapeDtypeStruct((Q, 1), jnp.int32),
        scratch_shapes=[
            pltpu.VMEM((Q, 1), jnp.float32),
            pltpu.VMEM((Q, 1), jnp.int32),
        ],
    )(p_head, *([g_head] * NSTREAMS))


def _gather_tails(g_tail, idx):
    info = plsc.get_sparse_core_info()
    nw = info.num_cores * info.num_subcores
    b_per_w = Q // nw

    dtp = g_tail.shape[1]

    @functools.partial(
        pl.kernel,
        mesh=plsc.VectorSubcoreMesh(core_axis_name="c", subcore_axis_name="s"),
        compiler_params=pltpu.CompilerParams(use_tc_tiling_on_sc=False),
        out_type=jax.ShapeDtypeStruct((Q, dtp), jnp.float32),
        scratch_types=[
            pltpu.VMEM((b_per_w,), jnp.int32),
            pltpu.VMEM((b_per_w, dtp), jnp.float32),
            pltpu.SemaphoreType.DMA,
        ],
    )
    def gather(table_hbm, idx_hbm, out_hbm, idx_v, rows_v, sem):
        wid = lax.axis_index("s") * info.num_cores + lax.axis_index("c")
        base = wid * b_per_w
        pltpu.sync_copy(idx_hbm.at[pl.ds(base, b_per_w)], idx_v)
        pltpu.async_copy(table_hbm.at[idx_v], rows_v, sem).wait()
        pltpu.sync_copy(rows_v, out_hbm.at[pl.ds(base, b_per_w)])

    return gather(g_tail, idx)


QBLK = 256


def _mlp_body(t_ref, w1_ref, b1_ref, w2_ref, b2_ref, out_ref):
    h = lax.dot_general(t_ref[...], w1_ref[...], (((1,), (1,)), ((), ())),
                        preferred_element_type=jnp.float32)
    h = jnp.maximum(h + b1_ref[...], 0.0)
    o = lax.dot_general(h, w2_ref[...], (((1,), (0,)), ((), ())),
                        preferred_element_type=jnp.float32)
    out_ref[...] = o + b2_ref[...]


def _mlp(tail, W1, b1, W2bd, b2):
    dt = tail.shape[1]
    return pl.pallas_call(
        _mlp_body,
        grid=(Q // QBLK,),
        in_specs=[
            pl.BlockSpec((QBLK, dt), lambda i: (i, 0)),
            pl.BlockSpec((MID, dt), lambda i: (0, 0)),
            pl.BlockSpec((1, MID), lambda i: (0, 0)),
            pl.BlockSpec((MID, OUT), lambda i: (0, 0)),
            pl.BlockSpec((1, OUT), lambda i: (0, 0)),
        ],
        out_specs=pl.BlockSpec((QBLK, OUT), lambda i: (i, 0)),
        out_shape=jax.ShapeDtypeStruct((Q, OUT), jnp.float32),
    )(tail, W1, b1.reshape(1, MID), W2bd, b2.reshape(1, OUT))


DTP = 304


def kernel(p_head, p_rel, g_head, g_rel, g_tail, W1, b1, W2, b2):
    an = p_head / (jnp.linalg.norm(p_head, axis=1, keepdims=True) + 1e-8)
    idx = _top1_index(an, g_head).reshape(Q)
    tail = _gather_tails(jnp.pad(g_tail, ((0, 0), (0, DTP - DT))), idx)
    W1 = jnp.pad(W1, ((0, 0), (0, DTP - DT)))
    w2t = jnp.transpose(W2, (0, 2, 1))
    gi = jnp.arange(GROUPS)
    W2bd = (jnp.zeros((GROUPS, MID // GROUPS, GROUPS, OUT // GROUPS), W2.dtype)
            .at[gi, :, gi, :].set(w2t)
            .reshape(MID, OUT))
    return _mlp(tail, W1, b1, W2bd, b2)


# speedup vs baseline: 1.0704x; 1.0704x over previous
import functools

import jax
import jax.numpy as jnp
from jax import lax
from jax.experimental import pallas as pl
from jax.experimental.pallas import tpu as pltpu
from jax.experimental.pallas import tpu_sc as plsc

Q, K, D, DT = 1024, 100000, 1024, 300
GROUPS, MID, OUT = 64, 4096, 1024
KBLK = 2000
NBLK = K // KBLK
NSTREAMS = 2
NSTEPS = NBLK // NSTREAMS


def _argmax_body(p_ref, *refs):
    g_refs = refs[:NSTREAMS]
    idx_ref = refs[NSTREAMS]
    maxv, argv = refs[NSTREAMS + 1:]
    i = pl.program_id(0)

    @pl.when(i == 0)
    def _():
        maxv[...] = jnp.full_like(maxv[...], -jnp.inf)
        argv[...] = jnp.zeros_like(argv[...])

    praw = p_ref[...]
    pss = jnp.sum(praw * praw, axis=1, keepdims=True)
    p = praw * (1.0 / (jnp.sqrt(pss) + 1e-8))
    best_m = None
    best_a = None
    for j, g_ref in enumerate(g_refs):
        g = g_ref[...]
        ss = jnp.sum(g * g, axis=1, keepdims=True)
        gn = g * (1.0 / (jnp.sqrt(ss) + 1e-8))
        s = lax.dot_general(p, gn, (((1,), (1,)), ((), ())),
                            preferred_element_type=jnp.float32)
        col = lax.broadcasted_iota(jnp.int32, s.shape, 1)
        m = jnp.max(s, axis=1, keepdims=True)
        cand = jnp.where(s == m, col, K)
        a = (jnp.min(cand, axis=1, keepdims=True)
             + (i * NSTREAMS + j) * KBLK)
        if best_m is None:
            best_m, best_a = m, a
        else:
            take = m > best_m
            best_m = jnp.where(take, m, best_m)
            best_a = jnp.where(take, a, best_a)
    upd = best_m > maxv[...]
    maxv[...] = jnp.where(upd, best_m, maxv[...])
    argv[...] = jnp.where(upd, best_a, argv[...])
    idx_ref[...] = argv[...]


def _top1_index(p_head, g_head):
    g_specs = [
        pl.BlockSpec((KBLK, D), lambda i, j=j: (NSTREAMS * i + j, 0))
        for j in range(NSTREAMS)
    ]
    return pl.pallas_call(
        _argmax_body,
        grid=(NSTEPS,),
        in_specs=[pl.BlockSpec((Q, D), lambda i: (0, 0))] + g_specs,
        out_specs=pl.BlockSpec((Q, 1), lambda i: (0, 0)),
        out_shape=jax.ShapeDtypeStruct((Q, 1), jnp.int32),
        scratch_shapes=[
            pltpu.VMEM((Q, 1), jnp.float32),
            pltpu.VMEM((Q, 1), jnp.int32),
        ],
    )(p_head, *([g_head] * NSTREAMS))


def _gather_tails(g_tail, idx):
    info = plsc.get_sparse_core_info()
    nw = info.num_cores * info.num_subcores
    b_per_w = Q // nw

    dtp = g_tail.shape[1]

    @functools.partial(
        pl.kernel,
        mesh=plsc.VectorSubcoreMesh(core_axis_name="c", subcore_axis_name="s"),
        compiler_params=pltpu.CompilerParams(use_tc_tiling_on_sc=False),
        out_type=jax.ShapeDtypeStruct((Q, dtp), jnp.float32),
        scratch_types=[
            pltpu.VMEM((b_per_w,), jnp.int32),
            pltpu.VMEM((b_per_w, dtp), jnp.float32),
            pltpu.SemaphoreType.DMA,
        ],
    )
    def gather(table_hbm, idx_hbm, out_hbm, idx_v, rows_v, sem):
        wid = lax.axis_index("s") * info.num_cores + lax.axis_index("c")
        base = wid * b_per_w
        pltpu.sync_copy(idx_hbm.at[pl.ds(base, b_per_w)], idx_v)
        pltpu.async_copy(table_hbm.at[idx_v], rows_v, sem).wait()
        pltpu.sync_copy(rows_v, out_hbm.at[pl.ds(base, b_per_w)])

    return gather(g_tail, idx)


GO = OUT // GROUPS
GI = MID // GROUPS


def _mlp_body(t_ref, w1_ref, b1_ref, w2_ref, b2_ref, out_ref):
    h = lax.dot_general(t_ref[...], w1_ref[...], (((1,), (1,)), ((), ())),
                        preferred_element_type=jnp.float32)
    h = jnp.maximum(h + b1_ref[...], 0.0)
    for g in range(GROUPS):
        og = lax.dot_general(h[:, g * GI:(g + 1) * GI],
                             w2_ref[g * GO:(g + 1) * GO, :],
                             (((1,), (1,)), ((), ())),
                             preferred_element_type=jnp.float32)
        out_ref[:, g * GO:(g + 1) * GO] = og + b2_ref[:, g * GO:(g + 1) * GO]


def _mlp(tail, W1, b1, W2r, b2):
    dt = tail.shape[1]
    return pl.pallas_call(
        _mlp_body,
        grid=(1,),
        in_specs=[
            pl.BlockSpec((Q, dt), lambda i: (0, 0)),
            pl.BlockSpec((MID, dt), lambda i: (0, 0)),
            pl.BlockSpec((1, MID), lambda i: (0, 0)),
            pl.BlockSpec((GROUPS * GO, GI), lambda i: (0, 0)),
            pl.BlockSpec((1, OUT), lambda i: (0, 0)),
        ],
        out_specs=pl.BlockSpec((Q, OUT), lambda i: (0, 0)),
        out_shape=jax.ShapeDtypeStruct((Q, OUT), jnp.float32),
    )(tail, W1, b1.reshape(1, MID), W2r, b2.reshape(1, OUT))


DTP = 304


def kernel(p_head, p_rel, g_head, g_rel, g_tail, W1, b1, W2, b2):
    idx = _top1_index(p_head, g_head).reshape(Q)
    tail = _gather_tails(jnp.pad(g_tail, ((0, 0), (0, DTP - DT))), idx)
    W1 = jnp.pad(W1, ((0, 0), (0, DTP - DT)))
    return _mlp(tail, W1, b1, W2.reshape(GROUPS * GO, GI), b2)


# speedup vs baseline: 1.6764x; 1.5661x over previous
import functools

import jax
import jax.numpy as jnp
from jax import lax
from jax.experimental import pallas as pl
from jax.experimental.pallas import tpu as pltpu
from jax.experimental.pallas import tpu_sc as plsc

Q, K, D, DT = 1024, 100000, 1024, 300
GROUPS, MID, OUT = 64, 4096, 1024
KBLK = 2000
NBLK = K // KBLK
NSTREAMS = 2
NSTEPS = NBLK // NSTREAMS


def _argmax_body(p_ref, *refs):
    g_refs = refs[:NSTREAMS]
    idx_ref = refs[NSTREAMS]
    maxv, argv = refs[NSTREAMS + 1:]
    i = pl.program_id(0)

    @pl.when(i == 0)
    def _():
        maxv[...] = jnp.full_like(maxv[...], -jnp.inf)
        argv[...] = jnp.zeros_like(argv[...])

    praw = p_ref[...]
    pss = jnp.sum(praw * praw, axis=1, keepdims=True)
    p = praw * (1.0 / (jnp.sqrt(pss) + 1e-8))
    best_m = None
    best_a = None
    for j, g_ref in enumerate(g_refs):
        g = g_ref[...]
        ss = jnp.sum(g * g, axis=1, keepdims=True)
        gn = g * (1.0 / (jnp.sqrt(ss) + 1e-8))
        s = lax.dot_general(p, gn, (((1,), (1,)), ((), ())),
                            preferred_element_type=jnp.float32)
        col = lax.broadcasted_iota(jnp.int32, s.shape, 1)
        m = jnp.max(s, axis=1, keepdims=True)
        cand = jnp.where(s == m, col, K)
        a = (jnp.min(cand, axis=1, keepdims=True)
             + (i * NSTREAMS + j) * KBLK)
        if best_m is None:
            best_m, best_a = m, a
        else:
            take = m > best_m
            best_m = jnp.where(take, m, best_m)
            best_a = jnp.where(take, a, best_a)
    upd = best_m > maxv[...]
    maxv[...] = jnp.where(upd, best_m, maxv[...])
    argv[...] = jnp.where(upd, best_a, argv[...])
    idx_ref[...] = argv[...]


def _top1_index(p_head, g_head):
    g_specs = [
        pl.BlockSpec((KBLK, D), lambda i, j=j: (NSTREAMS * i + j, 0))
        for j in range(NSTREAMS)
    ]
    return pl.pallas_call(
        _argmax_body,
        grid=(NSTEPS,),
        in_specs=[pl.BlockSpec((Q, D), lambda i: (0, 0))] + g_specs,
        out_specs=pl.BlockSpec((Q, 1), lambda i: (0, 0)),
        out_shape=jax.ShapeDtypeStruct((Q, 1), jnp.int32),
        scratch_shapes=[
            pltpu.VMEM((Q, 1), jnp.float32),
            pltpu.VMEM((Q, 1), jnp.int32),
        ],
    )(p_head, *([g_head] * NSTREAMS))


NSEM = 16


def _gather_body(idx_ref, table_ref, out_ref, sems):
    def issue(i):
        return pltpu.make_async_copy(
            table_ref.at[pl.ds(idx_ref[i], 1), :],
            out_ref.at[pl.ds(i, 1), :],
            sems.at[lax.rem(i, NSEM)],
        )

    def body(i, carry):
        @pl.when(i >= NSEM)
        def _():
            issue(i - NSEM).wait()

        issue(i).start()
        return carry

    lax.fori_loop(0, Q, body, 0)

    def drain(i, carry):
        issue(Q - NSEM + i).wait()
        return carry

    lax.fori_loop(0, NSEM, drain, 0)


def _gather_tails(g_tail, idx):
    dt = g_tail.shape[1]
    return pl.pallas_call(
        _gather_body,
        grid_spec=pltpu.PrefetchScalarGridSpec(
            num_scalar_prefetch=1,
            grid=(1,),
            in_specs=[pl.BlockSpec(memory_space=pl.ANY)],
            out_specs=pl.BlockSpec((Q, dt), lambda i, idx_ref: (0, 0)),
            scratch_shapes=[pltpu.SemaphoreType.DMA((NSEM,))],
        ),
        out_shape=jax.ShapeDtypeStruct((Q, dt), jnp.float32),
    )(idx, g_tail)


GO = OUT // GROUPS
GI = MID // GROUPS


def _mlp_body(t_ref, w1_ref, b1_ref, w2_ref, b2_ref, out_ref):
    h = lax.dot_general(t_ref[...], w1_ref[...], (((1,), (1,)), ((), ())),
                        preferred_element_type=jnp.float32)
    h = jnp.maximum(h + b1_ref[...], 0.0)
    for g in range(GROUPS):
        og = lax.dot_general(h[:, g * GI:(g + 1) * GI],
                             w2_ref[g * GO:(g + 1) * GO, :],
                             (((1,), (1,)), ((), ())),
                             preferred_element_type=jnp.float32)
        out_ref[:, g * GO:(g + 1) * GO] = og + b2_ref[:, g * GO:(g + 1) * GO]


def _mlp(tail, W1, b1, W2r, b2):
    dt = tail.shape[1]
    return pl.pallas_call(
        _mlp_body,
        grid=(1,),
        in_specs=[
            pl.BlockSpec((Q, dt), lambda i: (0, 0)),
            pl.BlockSpec((MID, dt), lambda i: (0, 0)),
            pl.BlockSpec((1, MID), lambda i: (0, 0)),
            pl.BlockSpec((GROUPS * GO, GI), lambda i: (0, 0)),
            pl.BlockSpec((1, OUT), lambda i: (0, 0)),
        ],
        out_specs=pl.BlockSpec((Q, OUT), lambda i: (0, 0)),
        out_shape=jax.ShapeDtypeStruct((Q, OUT), jnp.float32),
    )(tail, W1, b1.reshape(1, MID), W2r, b2.reshape(1, OUT))


DTP = 304


def kernel(p_head, p_rel, g_head, g_rel, g_tail, W1, b1, W2, b2):
    idx = _top1_index(p_head, g_head).reshape(Q)
    tail = _gather_tails(g_tail, idx)
    return _mlp(tail, W1, b1, W2.reshape(GROUPS * GO, GI), b2)
